# baseline (speedup 1.0000x reference)
import functools

import jax
import jax.numpy as jnp
from jax import lax
from jax.experimental import pallas as pl
from jax.experimental.pallas import tpu as pltpu
from jax.experimental.pallas import tpu_sc as plsc

H = 16
N = 577
NN = N * N
NUM_REL = 2212
TBL = NUM_REL * H
NW = 32
L = 16
CPW = 10416
P = NW * CPW
BH = 256
CB = 8192


@functools.cache
def _sc_gather_bias_fn():
    @functools.partial(
        pl.kernel,
        mesh=plsc.VectorSubcoreMesh(core_axis_name="c", subcore_axis_name="s"),
        out_type=jax.ShapeDtypeStruct((H * P,), jnp.float32),
        scratch_types=[
            pltpu.VMEM((TBL,), jnp.float32),
            pltpu.VMEM((CPW,), jnp.int32),
            pltpu.VMEM((CPW,), jnp.float32),
        ],
        compiler_params=pltpu.CompilerParams(needs_layout_passes=False),
    )
    def _sc_gather_bias(tbl_hbm, idx_hbm, out_hbm, tbl_v, idx_v, row_v):
        wid = lax.axis_index("s") * 2 + lax.axis_index("c")
        base = wid * CPW
        pltpu.sync_copy(tbl_hbm, tbl_v)
        pltpu.sync_copy(idx_hbm.at[pl.ds(base, CPW)], idx_v)

        def _scale(j, carry):
            v = idx_v[pl.ds(j * L, L)]
            idx_v[pl.ds(j * L, L)] = v * H
            return carry

        lax.fori_loop(0, CPW // L, _scale, 0)

        def _per_head(h, carry):
            def _gather(j, c2):
                g = plsc.load_gather(tbl_v, [idx_v[pl.ds(j * L, L)] + h])
                row_v[pl.ds(j * L, L)] = g
                return c2

            lax.fori_loop(0, CPW // L, _gather, 0)
            off = pl.multiple_of(h * P + base, 8)
            pltpu.sync_copy(row_v, out_hbm.at[pl.ds(off, CPW)])
            return carry

        lax.fori_loop(0, H, _per_head, 0)

    return _sc_gather_bias


def _add_body(a_ref, b_ref, o_ref):
    o_ref[...] = a_ref[...] + b_ref[...]


def kernel(attn, relative_position_bias_table, relative_position_index):
    tbl_flat = relative_position_bias_table.reshape(-1)
    idx_flat = relative_position_index.reshape(-1).astype(jnp.int32)
    idx_pad = jnp.pad(idx_flat, (0, P - NN))

    bias_t = _sc_gather_bias_fn()(tbl_flat, idx_pad).reshape(H, P)

    attn2 = attn.reshape(BH, NN)
    nblk = pl.cdiv(NN, CB)
    out2 = pl.pallas_call(
        _add_body,
        grid=(2, nblk, BH // H),
        in_specs=[
            pl.BlockSpec((8, CB), lambda hg, c, b: (b * 2 + hg, c)),
            pl.BlockSpec((8, CB), lambda hg, c, b: (hg, c)),
        ],
        out_specs=pl.BlockSpec((8, CB), lambda hg, c, b: (b * 2 + hg, c)),
        out_shape=jax.ShapeDtypeStruct((BH, NN), jnp.float32),
    )(attn2, bias_t)
    return out2.reshape(attn.shape)


# speedup vs baseline: 1.1509x; 1.1509x over previous
import functools

import jax
import jax.numpy as jnp
from jax import lax
from jax.experimental import pallas as pl
from jax.experimental.pallas import tpu as pltpu
from jax.experimental.pallas import tpu_sc as plsc

B = 16
H = 16
N = 577
NUM_REL = 2212
TBL = NUM_REL * H
L = 16
NW = 32

NR = 584
NRG = NR // 8
NC = 640
CHUNK = 8 * N
CLOAD = 4640
IDXP = (NRG - 1) * CHUNK + CLOAD
NTASK = H * NRG
CJ = 37

RB = 64


@functools.cache
def _sc_gather_bias_fn():
    @functools.partial(
        pl.kernel,
        mesh=plsc.VectorSubcoreMesh(core_axis_name="c", subcore_axis_name="s"),
        out_type=jax.ShapeDtypeStruct((H, NR, NC), jnp.float32),
        scratch_types=[
            pltpu.VMEM((TBL,), jnp.float32),
            pltpu.VMEM((CLOAD,), jnp.int32),
            pltpu.VMEM((8, NC), jnp.float32),
        ],
        compiler_params=pltpu.CompilerParams(needs_layout_passes=False),
    )
    def _sc_gather_bias(tbl_hbm, idx_hbm, out_hbm, tbl_v, idx_v, row_v):
        wid = lax.axis_index("s") * 2 + lax.axis_index("c")
        start = wid * 36 + wid // 2
        count = 36 + (wid & 1)
        pltpu.sync_copy(tbl_hbm, tbl_v)

        def _task(t, carry):
            rg = t // H
            h = t % H
            off = pl.multiple_of(rg * CHUNK, 8)
            pltpu.sync_copy(idx_hbm.at[pl.ds(off, CLOAD)], idx_v)

            def _row(r, c1):
                def _col(j, c2):
                    v = idx_v[pl.ds(r * N + j * L, L)]
                    g = plsc.load_gather(tbl_v, [v * H + h])
                    row_v[r, pl.ds(j * L, L)] = g
                    return c2

                lax.fori_loop(0, CJ, _col, c1)
                return c1

            lax.fori_loop(0, 8, _row, 0)
            pltpu.sync_copy(row_v, out_hbm.at[h, pl.ds(rg * 8, 8), :])
            return carry

        lax.fori_loop(start, start + count, _task, 0)

    return _sc_gather_bias


def _add_body(a_ref, b_ref, o_ref):
    o_ref[...] = a_ref[...] + b_ref[:, :, :N]


def kernel(attn, relative_position_bias_table, relative_position_index):
    tbl_flat = relative_position_bias_table.reshape(-1)
    idx_flat = relative_position_index.reshape(-1).astype(jnp.int32)
    idx_pad = jnp.pad(idx_flat, (0, IDXP - N * N))

    bias = _sc_gather_bias_fn()(tbl_flat, idx_pad)

    nrb = pl.cdiv(N, RB)
    out = pl.pallas_call(
        _add_body,
        grid=(H, nrb, B),
        in_specs=[
            pl.BlockSpec((1, 1, RB, N), lambda h, r, b: (b, h, r, 0)),
            pl.BlockSpec((1, RB, NC), lambda h, r, b: (h, r, 0)),
        ],
        out_specs=pl.BlockSpec((1, 1, RB, N), lambda h, r, b: (b, h, r, 0)),
        out_shape=jax.ShapeDtypeStruct((B, H, N, N), jnp.float32),
    )(attn, bias)
    return out


# speedup vs baseline: 2.7618x; 2.3996x over previous
import functools

import jax
import jax.numpy as jnp
from jax import lax
from jax.experimental import pallas as pl
from jax.experimental.pallas import tpu as pltpu
from jax.experimental.pallas import tpu_sc as plsc

B = 16
H = 16
N = 577
NUM_REL = 2212
TBL = NUM_REL * H
L = 16
NW = 32

NR = 584
NRG = NR // 8
NC = 640
CHUNK = 8 * N
CLOAD = 4640
IDXP = (NRG - 1) * CHUNK + CLOAD
NTASK = H * NRG
CJ = 37


@functools.cache
def _sc_gather_bias_fn():
    @functools.partial(
        pl.kernel,
        mesh=plsc.VectorSubcoreMesh(core_axis_name="c", subcore_axis_name="s"),
        out_type=jax.ShapeDtypeStruct((H, NR, NC), jnp.float32),
        scratch_types=[
            pltpu.VMEM((TBL,), jnp.float32),
            pltpu.VMEM((CLOAD,), jnp.int32),
            pltpu.VMEM((8, NC), jnp.float32),
        ],
        compiler_params=pltpu.CompilerParams(needs_layout_passes=False),
    )
    def _sc_gather_bias(tbl_hbm, idx_hbm, out_hbm, tbl_v, idx_v, row_v):
        wid = lax.axis_index("s") * 2 + lax.axis_index("c")
        start = wid * 36 + wid // 2
        count = 36 + (wid & 1)
        pltpu.sync_copy(tbl_hbm, tbl_v)

        def _task(t, carry):
            rg = t // H
            h = t % H

            @pl.when((h == 0) | (t == start))
            def _load_idx():
                off = pl.multiple_of(rg * CHUNK, 8)
                pltpu.sync_copy(idx_hbm.at[pl.ds(off, CLOAD)], idx_v)

                def _scale(j, c1):
                    v = idx_v[pl.ds(j * L, L)]
                    idx_v[pl.ds(j * L, L)] = v * H
                    return c1

                lax.fori_loop(0, CLOAD // L, _scale, 0)

            def _row(r, c1):
                base = r * N
                for j in range(CJ):
                    v = idx_v[pl.ds(base + j * L, L)]
                    g = plsc.load_gather(tbl_v, [v + h])
                    row_v[r, pl.ds(j * L, L)] = g
                return c1

            lax.fori_loop(0, 8, _row, 0)
            pltpu.sync_copy(row_v, out_hbm.at[h, pl.ds(rg * 8, 8), :])
            return carry

        lax.fori_loop(start, start + count, _task, 0)

    return _sc_gather_bias


def _add_body(a_ref, b_ref, o_ref):
    o_ref[...] = a_ref[...] + b_ref[:, :N, :N]


def kernel(attn, relative_position_bias_table, relative_position_index):
    tbl_flat = relative_position_bias_table.reshape(-1)
    idx_flat = relative_position_index.reshape(-1).astype(jnp.int32)
    idx_pad = jnp.pad(idx_flat, (0, IDXP - N * N))

    bias = _sc_gather_bias_fn()(tbl_flat, idx_pad)

    out = pl.pallas_call(
        _add_body,
        grid=(H, B),
        in_specs=[
            pl.BlockSpec((1, 1, N, N), lambda h, b: (b, h, 0, 0)),
            pl.BlockSpec((1, NR, NC), lambda h, b: (h, 0, 0)),
        ],
        out_specs=pl.BlockSpec((1, 1, N, N), lambda h, b: (b, h, 0, 0)),
        out_shape=jax.ShapeDtypeStruct((B, H, N, N), jnp.float32),
    )(attn, bias)
    return out
